# Initial kernel scaffold; baseline (speedup 1.0000x reference)
import jax
import jax.numpy as jnp
from jax.experimental import pallas as pl


def kernel(x, embed_table, W, b):
    raise NotImplementedError("write your pallas kernel here")


# speedup vs baseline: 5.8851x; 5.8851x over previous
import functools

import jax
import jax.numpy as jnp
from jax import lax
from jax.experimental import pallas as pl
from jax.experimental.pallas import tpu as pltpu
from jax.experimental.pallas import tpu_sc as plsc

VOCAB = 100000
D = 128
B = 4096
S = 50
C = 9
CP = 16
NW = 32
BPW = B // NW
VCHUNK = 2000


def _proj_body(t_ref, w_ref, o_ref):
    o_ref[...] = jnp.dot(t_ref[...], w_ref[...],
                         preferred_element_type=jnp.float32)


def _project(table, wp):
    return pl.pallas_call(
        _proj_body,
        grid=(VOCAB // VCHUNK,),
        in_specs=[
            pl.BlockSpec((VCHUNK, D), lambda i: (i, 0)),
            pl.BlockSpec((D, CP), lambda i: (0, 0)),
        ],
        out_specs=pl.BlockSpec((VCHUNK, CP), lambda i: (i, 0)),
        out_shape=jax.ShapeDtypeStruct((VOCAB, CP), jnp.float32),
    )(table, wp)


_MESH = plsc.VectorSubcoreMesh(core_axis_name="c", subcore_axis_name="s")


@functools.partial(
    pl.kernel,
    mesh=_MESH,
    compiler_params=pltpu.CompilerParams(use_tc_tiling_on_sc=False),
    out_type=jax.ShapeDtypeStruct((B, CP), jnp.float32),
    scratch_types=[
        pltpu.VMEM((S, BPW), jnp.int32),
        pltpu.VMEM((BPW, CP), jnp.float32),
        pltpu.VMEM((BPW, CP), jnp.float32),
        pltpu.VMEM((BPW,), jnp.int32),
        pltpu.VMEM((CP,), jnp.float32),
        pltpu.VMEM_SHARED((16 * BPW, CP), jnp.float32),
        pltpu.SemaphoreType.DMA,
    ],
)
def _pool(xr_hbm, p_hbm, bias_hbm, out_hbm,
          idx_v, rows_v, res_v, dst_v, bias_v, acc_sh, sem):
    cid = lax.axis_index("c")
    sid = lax.axis_index("s")
    wid = sid * 2 + cid
    obase = wid * BPW
    abase = sid * BPW

    pltpu.sync_copy(xr_hbm.at[wid], idx_v)
    pltpu.sync_copy(bias_hbm, bias_v)
    bvec = bias_v[...]

    for j in range(BPW):
        res_v[j, :] = bvec
    iota16 = lax.iota(jnp.int32, CP)
    for j in range(BPW // CP):
        dst_v[pl.ds(j * CP, CP)] = abase + j * CP + iota16

    pltpu.sync_copy(res_v, acc_sh.at[pl.ds(abase, BPW)])

    def body(s, carry):
        pltpu.async_copy(p_hbm.at[idx_v.at[s]], rows_v, sem).wait()
        pltpu.sync_copy(rows_v, acc_sh.at[dst_v], add=True)
        return carry

    lax.fori_loop(0, S, body, 0)

    pltpu.sync_copy(acc_sh.at[pl.ds(abase, BPW)], res_v)
    pltpu.sync_copy(res_v, out_hbm.at[pl.ds(obase, BPW)])


def kernel(x, embed_table, W, b):
    wp = jnp.pad(W, ((0, 0), (0, CP - C))) * (1.0 / S)
    p = _project(embed_table, wp)
    xr = x.reshape(NW, BPW, S).transpose(0, 2, 1)
    b16 = jnp.pad(b, (0, CP - C))
    out16 = _pool(xr, p, b16)
    return out16[:, :C]


# speedup vs baseline: 6.3922x; 1.0862x over previous
import functools

import jax
import jax.numpy as jnp
from jax import lax
from jax.experimental import pallas as pl
from jax.experimental.pallas import tpu as pltpu
from jax.experimental.pallas import tpu_sc as plsc

VOCAB = 100000
D = 128
B = 4096
S = 50
C = 9
CP = 16
NW = 32
BPW = B // NW
NG = BPW // 2
GP = 2 * S
GPP = 104
VCHUNK = 5000


def _proj_body(t_ref, w_ref, o_ref):
    o_ref[...] = jnp.dot(t_ref[...], w_ref[...],
                         preferred_element_type=jnp.float32)


def _project(table, wp):
    return pl.pallas_call(
        _proj_body,
        grid=(VOCAB // VCHUNK,),
        in_specs=[
            pl.BlockSpec((VCHUNK, D), lambda i: (i, 0)),
            pl.BlockSpec((D, CP), lambda i: (0, 0)),
        ],
        out_specs=pl.BlockSpec((VCHUNK, CP), lambda i: (i, 0)),
        out_shape=jax.ShapeDtypeStruct((VOCAB, CP), jnp.float32),
    )(table, wp)


_MESH = plsc.VectorSubcoreMesh(core_axis_name="c", subcore_axis_name="s")


@functools.partial(
    pl.kernel,
    mesh=_MESH,
    compiler_params=pltpu.CompilerParams(use_tc_tiling_on_sc=False),
    out_type=jax.ShapeDtypeStruct((B, CP), jnp.float32),
    scratch_types=[
        pltpu.VMEM((NG, GPP), jnp.int32),
        pltpu.VMEM((GPP, CP), jnp.float32),
        pltpu.VMEM((GPP, CP), jnp.float32),
        pltpu.VMEM((BPW, CP), jnp.float32),
        pltpu.VMEM((CP,), jnp.float32),
        pltpu.SemaphoreType.DMA,
        pltpu.SemaphoreType.DMA,
    ],
)
def _pool(xg_hbm, p_hbm, bias_hbm, out_hbm,
          idx_v, rows0_v, rows1_v, out_v, bias_v, sem0, sem1):
    cid = lax.axis_index("c")
    sid = lax.axis_index("s")
    wid = sid * 2 + cid
    obase = wid * BPW

    pltpu.sync_copy(xg_hbm.at[wid], idx_v)
    pltpu.sync_copy(bias_hbm, bias_v)
    bvec = bias_v[...]

    bufs = ((rows0_v, sem0), (rows1_v, sem1))

    pltpu.async_copy(p_hbm.at[idx_v.at[0]], rows0_v, sem0)
    pltpu.async_copy(p_hbm.at[idx_v.at[1]], rows1_v, sem1)

    def accumulate(rows_v, g):
        for r in range(2):
            p0 = bvec + rows_v[r * S, :]
            p1 = rows_v[r * S + 1, :]
            for k in range(2, S, 2):
                p0 = p0 + rows_v[r * S + k, :]
                p1 = p1 + rows_v[r * S + k + 1, :]
            out_v[2 * g + r, :] = p0 + p1

    def body(i, carry):
        for b, (rows_v, sem) in enumerate(bufs):
            g = 2 * i + b
            pltpu.make_async_copy(p_hbm.at[idx_v.at[g]], rows_v, sem).wait()
            accumulate(rows_v, g)

            @pl.when(g + 2 < NG)
            def _():
                pltpu.async_copy(p_hbm.at[idx_v.at[g + 2]], rows_v, sem)

        return carry

    lax.fori_loop(0, NG // 2, body, 0)

    pltpu.sync_copy(out_v, out_hbm.at[pl.ds(obase, BPW)])


def kernel(x, embed_table, W, b):
    wp = jnp.pad(W, ((0, 0), (0, CP - C))) * (1.0 / S)
    p = _project(embed_table, wp)
    xg = jnp.pad(x.reshape(NW, NG, GP), ((0, 0), (0, 0), (0, GPP - GP)))
    b16 = jnp.pad(b, (0, CP - C))
    out16 = _pool(xg, p, b16)
    return out16[:, :C]
